# Initial kernel scaffold; baseline (speedup 1.0000x reference)
import jax
import jax.numpy as jnp
from jax.experimental import pallas as pl


def kernel(h_current, history_db, alpha):
    raise NotImplementedError("write your pallas kernel here")


# speedup vs baseline: 2.9802x; 2.9802x over previous
import functools

import jax
import jax.numpy as jnp
from jax import lax
from jax.experimental import pallas as pl
from jax.experimental.pallas import tpu as pltpu
from jax.experimental.pallas import tpu_sc as plsc

B = 1024
D = 512
N = 100000
K = 16
W = 2000
NB = N // W

NW = 32
QW = B // NW
CH = D // 16


def _topk_body(hc_ref, db_ref, tv_ref, ti_ref, mu_ref):
    j = pl.program_id(0)

    @pl.when(j == 0)
    def _init():
        tv_ref[...] = jnp.full((B, K), -jnp.inf, dtype=jnp.float32)
        ti_ref[...] = jnp.zeros((B, K), dtype=jnp.int32)

    h = hc_ref[...]
    hn = h * lax.rsqrt(jnp.maximum(jnp.sum(h * h, axis=1, keepdims=True), 1e-24))
    db = db_ref[...]
    inv = lax.rsqrt(jnp.maximum(jnp.sum(db * db, axis=1), 1e-24))
    s = lax.dot_general(hn, db, (((1,), (1,)), ((), ())),
                        preferred_element_type=jnp.float32)
    s = s * inv[None, :]

    colid = lax.broadcasted_iota(jnp.int32, (B, W), 1)
    lane = lax.broadcasted_iota(jnp.int32, (B, K), 1)
    tv0 = tv_ref[...]
    ti0 = ti_ref[...]

    def cond_fn(carry):
        return carry[0]

    def body_fn(carry):
        _, vprev, tv, ti = carry
        sm = jnp.where(s < vprev, s, -jnp.inf)
        v = jnp.max(sm, axis=1, keepdims=True)
        thr = tv[:, K - 1:K]
        need = v > thr
        col = jnp.min(jnp.where(sm == v, colid, jnp.int32(N)),
                      axis=1, keepdims=True)
        gid = j * W + col
        rank = jnp.sum((tv >= v).astype(jnp.int32), axis=1, keepdims=True)
        sv = pltpu.roll(tv, shift=1, axis=1)
        si = pltpu.roll(ti, shift=1, axis=1)
        nv = jnp.where(lane < rank, tv, jnp.where(lane == rank, v, sv))
        ni = jnp.where(lane < rank, ti, jnp.where(lane == rank, gid, si))
        tv2 = jnp.where(need, nv, tv)
        ti2 = jnp.where(need, ni, ti)
        return jnp.any(need), v, tv2, ti2

    _, _, tvf, tif = lax.while_loop(
        cond_fn, body_fn,
        (jnp.bool_(True), jnp.full((B, 1), jnp.inf, dtype=jnp.float32), tv0, ti0))

    tv_ref[...] = tvf
    ti_ref[...] = tif

    @pl.when(j == NB - 1)
    def _fin():
        m = jnp.max(tvf, axis=1, keepdims=True)
        e = jnp.exp(tvf - m)
        mu_ref[...] = e / jnp.sum(e, axis=1, keepdims=True)


_topk = pl.pallas_call(
    _topk_body,
    grid=(NB,),
    in_specs=[
        pl.BlockSpec((B, D), lambda j: (0, 0)),
        pl.BlockSpec((W, D), lambda j: (j, 0)),
    ],
    out_specs=[
        pl.BlockSpec((B, K), lambda j: (0, 0)),
        pl.BlockSpec((B, K), lambda j: (0, 0)),
        pl.BlockSpec((B, K), lambda j: (0, 0)),
    ],
    out_shape=[
        jax.ShapeDtypeStruct((B, K), jnp.float32),
        jax.ShapeDtypeStruct((B, K), jnp.int32),
        jax.ShapeDtypeStruct((B, K), jnp.float32),
    ],
    compiler_params=pltpu.CompilerParams(dimension_semantics=("arbitrary",)),
)


def _gather_blend_body(db_hbm, ti_hbm, mu_hbm, hc_hbm, al_hbm, out_hbm,
                       ti_v, mu_v, hc_v, out_v, rows_v, al_v, sem):
    cid = lax.axis_index("c")
    sid = lax.axis_index("s")
    wid = sid * 2 + cid
    q0 = wid * QW
    pltpu.sync_copy(ti_hbm.at[pl.ds(q0, QW)], ti_v)
    pltpu.sync_copy(mu_hbm.at[pl.ds(q0, QW)], mu_v)
    pltpu.sync_copy(hc_hbm.at[pl.ds(q0, QW)], hc_v)
    pltpu.sync_copy(al_hbm, al_v)
    av = al_v[...]
    a = 1.0 / (1.0 + jnp.exp(-av))
    one_m_a = 1.0 - a

    def qloop(q, carry):
        pltpu.async_copy(db_hbm.at[ti_v.at[q]], rows_v, sem).wait()
        mks = [plsc.load_gather(mu_v.at[q], [jnp.full((16,), k, jnp.int32)])
               for k in range(K)]
        for c in range(CH):
            acc = mks[0] * rows_v[0, pl.ds(c * 16, 16)]
            for k in range(1, K):
                acc = acc + mks[k] * rows_v[k, pl.ds(c * 16, 16)]
            hcc = hc_v[q, pl.ds(c * 16, 16)]
            out_v[q, pl.ds(c * 16, 16)] = a * hcc + one_m_a * acc
        return carry

    lax.fori_loop(0, QW, qloop, 0)
    pltpu.sync_copy(out_v, out_hbm.at[pl.ds(q0, QW)])


@functools.cache
def _make_gather_blend():
    return pl.kernel(
        _gather_blend_body,
        out_type=jax.ShapeDtypeStruct((B, D), jnp.float32),
        mesh=plsc.VectorSubcoreMesh(core_axis_name="c", subcore_axis_name="s"),
        compiler_params=pltpu.CompilerParams(needs_layout_passes=False),
        scratch_types=[
            pltpu.VMEM((QW, K), jnp.int32),
            pltpu.VMEM((QW, K), jnp.float32),
            pltpu.VMEM((QW, D), jnp.float32),
            pltpu.VMEM((QW, D), jnp.float32),
            pltpu.VMEM((K, D), jnp.float32),
            pltpu.VMEM((16,), jnp.float32),
            pltpu.SemaphoreType.DMA,
        ],
    )


def kernel(h_current, history_db, alpha):
    tv, ti, mu = _topk(h_current, history_db)
    al = jnp.broadcast_to(jnp.reshape(alpha, (1,)).astype(jnp.float32), (16,))
    return _make_gather_blend()(history_db, ti, mu, h_current, al)


# speedup vs baseline: 2.9878x; 1.0026x over previous
import functools

import jax
import jax.numpy as jnp
from jax import lax
from jax.experimental import pallas as pl
from jax.experimental.pallas import tpu as pltpu
from jax.experimental.pallas import tpu_sc as plsc

B = 1024
D = 512
N = 100000
K = 16
W = 2000
NB = N // W

NW = 32
QW = B // NW
CH = D // 16


def _topk_body(hc_ref, db_ref, tv_ref, ti_ref, mu_ref):
    j = pl.program_id(0)

    @pl.when(j == 0)
    def _init():
        tv_ref[...] = jnp.full((B, K), -jnp.inf, dtype=jnp.float32)
        ti_ref[...] = jnp.zeros((B, K), dtype=jnp.int32)

    h = hc_ref[...]
    hn = h * lax.rsqrt(jnp.maximum(jnp.sum(h * h, axis=1, keepdims=True), 1e-24))
    db = db_ref[...]
    inv = lax.rsqrt(jnp.maximum(jnp.sum(db * db, axis=1), 1e-24))
    s = lax.dot_general(hn.astype(jnp.bfloat16), db.astype(jnp.bfloat16),
                        (((1,), (1,)), ((), ())),
                        preferred_element_type=jnp.float32)
    s = s * inv[None, :]

    colid = lax.broadcasted_iota(jnp.int32, (B, W), 1)
    lane = lax.broadcasted_iota(jnp.int32, (B, K), 1)
    tv0 = tv_ref[...]
    ti0 = ti_ref[...]

    def cond_fn(carry):
        return carry[0]

    def body_fn(carry):
        _, vprev, tv, ti = carry
        sm = jnp.where(s < vprev, s, -jnp.inf)
        v = jnp.max(sm, axis=1, keepdims=True)
        thr = tv[:, K - 1:K]
        need = v > thr
        col = jnp.min(jnp.where(sm == v, colid, jnp.int32(N)),
                      axis=1, keepdims=True)
        gid = j * W + col
        rank = jnp.sum((tv >= v).astype(jnp.int32), axis=1, keepdims=True)
        sv = pltpu.roll(tv, shift=1, axis=1)
        si = pltpu.roll(ti, shift=1, axis=1)
        nv = jnp.where(lane < rank, tv, jnp.where(lane == rank, v, sv))
        ni = jnp.where(lane < rank, ti, jnp.where(lane == rank, gid, si))
        tv2 = jnp.where(need, nv, tv)
        ti2 = jnp.where(need, ni, ti)
        return jnp.any(need), v, tv2, ti2

    _, _, tvf, tif = lax.while_loop(
        cond_fn, body_fn,
        (jnp.bool_(True), jnp.full((B, 1), jnp.inf, dtype=jnp.float32), tv0, ti0))

    tv_ref[...] = tvf
    ti_ref[...] = tif

    @pl.when(j == NB - 1)
    def _fin():
        m = jnp.max(tvf, axis=1, keepdims=True)
        e = jnp.exp(tvf - m)
        mu_ref[...] = e / jnp.sum(e, axis=1, keepdims=True)


_topk = pl.pallas_call(
    _topk_body,
    grid=(NB,),
    in_specs=[
        pl.BlockSpec((B, D), lambda j: (0, 0)),
        pl.BlockSpec((W, D), lambda j: (j, 0)),
    ],
    out_specs=[
        pl.BlockSpec((B, K), lambda j: (0, 0)),
        pl.BlockSpec((B, K), lambda j: (0, 0)),
        pl.BlockSpec((B, K), lambda j: (0, 0)),
    ],
    out_shape=[
        jax.ShapeDtypeStruct((B, K), jnp.float32),
        jax.ShapeDtypeStruct((B, K), jnp.int32),
        jax.ShapeDtypeStruct((B, K), jnp.float32),
    ],
    compiler_params=pltpu.CompilerParams(dimension_semantics=("arbitrary",)),
)


def _gather_blend_body(db_hbm, ti_hbm, mu_hbm, hc_hbm, al_hbm, out_hbm,
                       ti_v, mu_v, hc_v, out_v, rows_v, al_v, sem):
    cid = lax.axis_index("c")
    sid = lax.axis_index("s")
    wid = sid * 2 + cid
    q0 = wid * QW
    pltpu.sync_copy(ti_hbm.at[pl.ds(q0, QW)], ti_v)
    pltpu.sync_copy(mu_hbm.at[pl.ds(q0, QW)], mu_v)
    pltpu.sync_copy(hc_hbm.at[pl.ds(q0, QW)], hc_v)
    pltpu.sync_copy(al_hbm, al_v)
    av = al_v[...]
    a = 1.0 / (1.0 + jnp.exp(-av))
    one_m_a = 1.0 - a

    def qloop(q, carry):
        pltpu.async_copy(db_hbm.at[ti_v.at[q]], rows_v, sem).wait()
        mks = [plsc.load_gather(mu_v.at[q], [jnp.full((16,), k, jnp.int32)])
               for k in range(K)]
        for c in range(CH):
            acc = mks[0] * rows_v[0, pl.ds(c * 16, 16)]
            for k in range(1, K):
                acc = acc + mks[k] * rows_v[k, pl.ds(c * 16, 16)]
            hcc = hc_v[q, pl.ds(c * 16, 16)]
            out_v[q, pl.ds(c * 16, 16)] = a * hcc + one_m_a * acc
        return carry

    lax.fori_loop(0, QW, qloop, 0)
    pltpu.sync_copy(out_v, out_hbm.at[pl.ds(q0, QW)])


@functools.cache
def _make_gather_blend():
    return pl.kernel(
        _gather_blend_body,
        out_type=jax.ShapeDtypeStruct((B, D), jnp.float32),
        mesh=plsc.VectorSubcoreMesh(core_axis_name="c", subcore_axis_name="s"),
        compiler_params=pltpu.CompilerParams(needs_layout_passes=False),
        scratch_types=[
            pltpu.VMEM((QW, K), jnp.int32),
            pltpu.VMEM((QW, K), jnp.float32),
            pltpu.VMEM((QW, D), jnp.float32),
            pltpu.VMEM((QW, D), jnp.float32),
            pltpu.VMEM((K, D), jnp.float32),
            pltpu.VMEM((16,), jnp.float32),
            pltpu.SemaphoreType.DMA,
        ],
    )


def kernel(h_current, history_db, alpha):
    tv, ti, mu = _topk(h_current, history_db)
    al = jnp.broadcast_to(jnp.reshape(alpha, (1,)).astype(jnp.float32), (16,))
    return _make_gather_blend()(history_db, ti, mu, h_current, al)


# speedup vs baseline: 3.7707x; 1.2620x over previous
import functools

import jax
import jax.numpy as jnp
from jax import lax
from jax.experimental import pallas as pl
from jax.experimental.pallas import tpu as pltpu
from jax.experimental.pallas import tpu_sc as plsc

B = 1024
D = 512
N = 100000
K = 16
W = 2000
NB = N // W

NW = 32
QW = B // NW
CH = D // 16


def _topk_body(hc_ref, db_ref, tv_ref, ti_ref, mu_ref):
    j = pl.program_id(0)

    @pl.when(j == 0)
    def _init():
        tv_ref[...] = jnp.full((B, K), -jnp.inf, dtype=jnp.float32)
        ti_ref[...] = jnp.zeros((B, K), dtype=jnp.int32)

    h = hc_ref[...]
    hn = h * lax.rsqrt(jnp.maximum(jnp.sum(h * h, axis=1, keepdims=True), 1e-24))
    db = db_ref[...]
    inv = lax.rsqrt(jnp.maximum(jnp.sum(db * db, axis=1), 1e-24))
    s = lax.dot_general(hn.astype(jnp.bfloat16), db.astype(jnp.bfloat16),
                        (((1,), (1,)), ((), ())),
                        preferred_element_type=jnp.float32)
    s = s * inv[None, :]

    colid = lax.broadcasted_iota(jnp.int32, (B, W), 1)
    ubits = lax.bitcast_convert_type(s + 3.0, jnp.int32)
    key = (ubits & jnp.int32(-2048)) | (2047 - colid)

    lane = lax.broadcasted_iota(jnp.int32, (B, K), 1)
    tv0 = tv_ref[...]
    ti0 = ti_ref[...]

    def cond_fn(carry):
        return carry[0]

    def body_fn(carry):
        _, vprev, tv, ti = carry
        vk = jnp.max(jnp.where(key < vprev, key, jnp.int32(0)),
                     axis=1, keepdims=True)
        v = lax.bitcast_convert_type(vk & jnp.int32(-2048),
                                     jnp.float32) - 3.0
        col = 2047 - (vk & jnp.int32(2047))
        thr = tv[:, K - 1:K]
        need = v > thr
        gid = j * W + col
        rank = jnp.sum((tv >= v).astype(jnp.int32), axis=1, keepdims=True)
        sv = pltpu.roll(tv, shift=1, axis=1)
        si = pltpu.roll(ti, shift=1, axis=1)
        nv = jnp.where(lane < rank, tv, jnp.where(lane == rank, v, sv))
        ni = jnp.where(lane < rank, ti, jnp.where(lane == rank, gid, si))
        tv2 = jnp.where(need, nv, tv)
        ti2 = jnp.where(need, ni, ti)
        return jnp.any(need), vk, tv2, ti2

    _, _, tvf, tif = lax.while_loop(
        cond_fn, body_fn,
        (jnp.bool_(True), jnp.full((B, 1), 0x7FFFFFFF, dtype=jnp.int32),
         tv0, ti0))

    tv_ref[...] = tvf
    ti_ref[...] = tif

    @pl.when(j == NB - 1)
    def _fin():
        m = jnp.max(tvf, axis=1, keepdims=True)
        e = jnp.exp(tvf - m)
        mu_ref[...] = e / jnp.sum(e, axis=1, keepdims=True)


_topk = pl.pallas_call(
    _topk_body,
    grid=(NB,),
    in_specs=[
        pl.BlockSpec((B, D), lambda j: (0, 0)),
        pl.BlockSpec((W, D), lambda j: (j, 0)),
    ],
    out_specs=[
        pl.BlockSpec((B, K), lambda j: (0, 0)),
        pl.BlockSpec((B, K), lambda j: (0, 0)),
        pl.BlockSpec((B, K), lambda j: (0, 0)),
    ],
    out_shape=[
        jax.ShapeDtypeStruct((B, K), jnp.float32),
        jax.ShapeDtypeStruct((B, K), jnp.int32),
        jax.ShapeDtypeStruct((B, K), jnp.float32),
    ],
    compiler_params=pltpu.CompilerParams(dimension_semantics=("arbitrary",)),
)


def _gather_blend_body(db_hbm, ti_hbm, mu_hbm, hc_hbm, al_hbm, out_hbm,
                       ti_v, mu_v, hc_v, out_v, rows_v, al_v, sem):
    cid = lax.axis_index("c")
    sid = lax.axis_index("s")
    wid = sid * 2 + cid
    q0 = wid * QW
    pltpu.sync_copy(ti_hbm.at[pl.ds(q0, QW)], ti_v)
    pltpu.sync_copy(mu_hbm.at[pl.ds(q0, QW)], mu_v)
    pltpu.sync_copy(hc_hbm.at[pl.ds(q0, QW)], hc_v)
    pltpu.sync_copy(al_hbm, al_v)
    av = al_v[...]
    a = 1.0 / (1.0 + jnp.exp(-av))
    one_m_a = 1.0 - a

    def qloop(q, carry):
        pltpu.async_copy(db_hbm.at[ti_v.at[q]], rows_v, sem).wait()
        mks = [plsc.load_gather(mu_v.at[q], [jnp.full((16,), k, jnp.int32)])
               for k in range(K)]
        for c in range(CH):
            acc = mks[0] * rows_v[0, pl.ds(c * 16, 16)]
            for k in range(1, K):
                acc = acc + mks[k] * rows_v[k, pl.ds(c * 16, 16)]
            hcc = hc_v[q, pl.ds(c * 16, 16)]
            out_v[q, pl.ds(c * 16, 16)] = a * hcc + one_m_a * acc
        return carry

    lax.fori_loop(0, QW, qloop, 0)
    pltpu.sync_copy(out_v, out_hbm.at[pl.ds(q0, QW)])


@functools.cache
def _make_gather_blend():
    return pl.kernel(
        _gather_blend_body,
        out_type=jax.ShapeDtypeStruct((B, D), jnp.float32),
        mesh=plsc.VectorSubcoreMesh(core_axis_name="c", subcore_axis_name="s"),
        compiler_params=pltpu.CompilerParams(needs_layout_passes=False),
        scratch_types=[
            pltpu.VMEM((QW, K), jnp.int32),
            pltpu.VMEM((QW, K), jnp.float32),
            pltpu.VMEM((QW, D), jnp.float32),
            pltpu.VMEM((QW, D), jnp.float32),
            pltpu.VMEM((K, D), jnp.float32),
            pltpu.VMEM((16,), jnp.float32),
            pltpu.SemaphoreType.DMA,
        ],
    )


def kernel(h_current, history_db, alpha):
    tv, ti, mu = _topk(h_current, history_db)
    al = jnp.broadcast_to(jnp.reshape(alpha, (1,)).astype(jnp.float32), (16,))
    return _make_gather_blend()(history_db, ti, mu, h_current, al)
